# Initial kernel scaffold; baseline (speedup 1.0000x reference)
import jax
import jax.numpy as jnp
from jax.experimental import pallas as pl


def kernel(input_ids, position_ids, token_embedding, position_embedding):
    raise NotImplementedError("write your pallas kernel here")


# speedup vs baseline: 1.4896x; 1.4896x over previous
import functools

import jax
import jax.numpy as jnp
from jax import lax
from jax.experimental import pallas as pl
from jax.experimental.pallas import tpu as pltpu
from jax.experimental.pallas import tpu_sc as plsc

VOCAB = 49408
N_WORDS = 77
D = 768
B = 1024

NW = 32
TOTAL = B * N_WORDS
PER_W = TOTAL // NW
CHUNK = 56
N_CHUNKS = PER_W // CHUNK
LANES = 16
D_SLICES = D // LANES


def _sc_embed(tok_ids, pos_ids, tok_emb, pos_emb):
    mesh = plsc.VectorSubcoreMesh(core_axis_name="c", subcore_axis_name="s")

    @functools.partial(
        pl.kernel,
        mesh=mesh,
        out_type=jax.ShapeDtypeStruct((TOTAL, D), jnp.float32),
        scratch_types=[
            pltpu.VMEM((N_CHUNKS, CHUNK), jnp.int32),
            pltpu.VMEM((N_CHUNKS, CHUNK), jnp.int32),
            pltpu.VMEM((CHUNK, D), jnp.float32),
            pltpu.VMEM((CHUNK, D), jnp.float32),
            pltpu.SemaphoreType.DMA,
            pltpu.SemaphoreType.DMA,
        ],
    )
    def k(tok_ids_hbm, pos_ids_hbm, tok_emb_hbm, pos_emb_hbm, out_hbm,
          tok_idx_v, pos_idx_v, tok_rows_v, pos_rows_v, sem_t, sem_p):
        wid = lax.axis_index("s") * 2 + lax.axis_index("c")
        base = wid * PER_W
        pltpu.sync_copy(tok_ids_hbm.at[wid], tok_idx_v)
        pltpu.sync_copy(pos_ids_hbm.at[wid], pos_idx_v)

        def chunk_body(c, carry):
            cp_t = pltpu.async_copy(
                tok_emb_hbm.at[tok_idx_v.at[c]], tok_rows_v, sem_t)
            cp_p = pltpu.async_copy(
                pos_emb_hbm.at[pos_idx_v.at[c]], pos_rows_v, sem_p)
            cp_t.wait()
            cp_p.wait()

            def row_body(r, carry2):
                for j in range(D_SLICES):
                    sl = pl.ds(j * LANES, LANES)
                    tok_rows_v[r, sl] = tok_rows_v[r, sl] + pos_rows_v[r, sl]
                return carry2

            lax.fori_loop(0, CHUNK, row_body, 0)
            pltpu.sync_copy(tok_rows_v,
                            out_hbm.at[pl.ds(base + c * CHUNK, CHUNK)])
            return carry

        lax.fori_loop(0, N_CHUNKS, chunk_body, 0)

    return k(tok_ids, pos_ids, tok_emb, pos_emb)


def kernel(input_ids, position_ids, token_embedding, position_embedding):
    tok_ids = input_ids.reshape(NW, N_CHUNKS, CHUNK).astype(jnp.int32)
    pos_ids = position_ids.reshape(NW, N_CHUNKS, CHUNK).astype(jnp.int32)
    out = _sc_embed(tok_ids, pos_ids, token_embedding, position_embedding)
    return out.reshape(B, N_WORDS, D)
